# baseline (speedup 1.0000x reference)
import functools

import jax
import jax.numpy as jnp
from jax import lax
from jax.experimental import pallas as pl
from jax.experimental.pallas import tpu as pltpu

BF16 = jnp.bfloat16
F32 = jnp.float32


def _rup(x, m):
    return (x + m - 1) // m * m


def _tapconv_kernel(xf, wt, sc, bi, o, xbuf, obuf, isem, osem,
                    *, mblk, offs, steps, mask_hw, lpack):
    j = pl.program_id(0)
    span = max(offs)
    rows = _rup(mblk + span, 8)
    base = j * steps

    def start_in(slot, gi):
        pltpu.make_async_copy(xf.at[pl.ds(gi * mblk, rows)],
                              xbuf.at[slot], isem.at[slot]).start()

    def wait_in(slot):
        pltpu.make_async_copy(xbuf.at[slot], xbuf.at[slot],
                              isem.at[slot]).wait()

    def start_out(slot, gi):
        pltpu.make_async_copy(obuf.at[slot],
                              o.at[pl.ds(gi * mblk, mblk)],
                              osem.at[slot]).start()

    def wait_out(slot):
        pltpu.make_async_copy(obuf.at[slot], obuf.at[slot],
                              osem.at[slot]).wait()

    start_in(0, base)

    def body(s, carry):
        cur = lax.rem(s, 2)
        nxt = lax.rem(s + 1, 2)

        @pl.when(s + 1 < steps)
        def _():
            start_in(nxt, base + s + 1)

        wait_in(cur)

        @pl.when(s >= 2)
        def _():
            wait_out(cur)

        xr = xbuf.at[cur]
        acc = None
        for t in range(len(offs) // lpack):
            if lpack == 1:
                xs = xr[pl.ds(offs[t], mblk), :]
            else:
                xs = jnp.concatenate(
                    [xr[pl.ds(offs[t * lpack + i], mblk), :]
                     for i in range(lpack)], axis=1)
            d = jnp.dot(xs, wt[t], preferred_element_type=F32)
            acc = d if acc is None else acc + d
        y = jnp.maximum(acc * sc[...] + bi[...], 0.0)
        if mask_hw is not None:
            colper, colvalid, rowper, rowvalid = mask_hw
            g = jax.lax.broadcasted_iota(jnp.int32, (mblk, 1), 0) \
                + (base + s) * mblk
            col = g % colper
            row = (g // colper) % rowper
            y = jnp.where((col < colvalid) & (row < rowvalid), y, 0.0)
        obuf.at[cur][...] = y.astype(obuf.dtype)
        start_out(cur, base + s)
        return carry

    lax.fori_loop(0, steps, body, 0)
    wait_out(lax.rem(steps - 2, 2))
    wait_out(lax.rem(steps - 1, 2))


def _tapconv(xflat, wt, scale, bias, *, offsets, mblk, mask_hw=None,
             lpack=1):
    N, K = xflat.shape
    ndots, _, cout = wt.shape
    assert ndots * lpack == len(offsets)
    mblk = min(mblk, max(N // 4 // 8 * 8, 8))
    lo = -min(offsets)
    offs = [o + lo for o in offsets]
    span = max(offs)
    ntot = _rup(N, 2 * mblk)
    steps = ntot // (2 * mblk)
    rows = _rup(mblk + span, 8)
    xp = jnp.pad(xflat, ((lo, (ntot - N) + (rows - mblk - lo)), (0, 0)))
    sc = scale.reshape(1, cout).astype(F32)
    bi = bias.reshape(1, cout).astype(F32)

    out = pl.pallas_call(
        functools.partial(_tapconv_kernel, mblk=mblk, offs=offs,
                          steps=steps, mask_hw=mask_hw, lpack=lpack),
        out_shape=jax.ShapeDtypeStruct((ntot, cout), BF16),
        grid_spec=pltpu.PrefetchScalarGridSpec(
            num_scalar_prefetch=0,
            grid=(2,),
            in_specs=[
                pl.BlockSpec(memory_space=pl.ANY),
                pl.BlockSpec(wt.shape, lambda j: (0, 0, 0)),
                pl.BlockSpec((1, cout), lambda j: (0, 0)),
                pl.BlockSpec((1, cout), lambda j: (0, 0)),
            ],
            out_specs=pl.BlockSpec(memory_space=pl.ANY),
            scratch_shapes=[
                pltpu.VMEM((2, rows, K), BF16),
                pltpu.VMEM((2, mblk, cout), BF16),
                pltpu.SemaphoreType.DMA((2,)),
                pltpu.SemaphoreType.DMA((2,)),
            ],
        ),
        compiler_params=pltpu.CompilerParams(
            dimension_semantics=("parallel",),
            vmem_limit_bytes=48 * 1024 * 1024,
        ),
    )(xp, wt, sc, bi)
    return out[:N]


def _pool_kernel(xr, o, *, C):
    m = jnp.maximum(xr[:, 0], xr[:, 1])
    o[...] = jnp.maximum(m[:, :, :C], m[:, :, C:])


def _maxpool(x):
    B, H, W, C = x.shape
    Ho, Wo = H // 2, W // 2
    rows = B * Ho
    x4 = x.reshape(rows, 2, Wo, 2 * C)
    R = 8
    for cand in (64, 32, 16):
        if rows % cand == 0:
            R = cand
            break
    out = pl.pallas_call(
        functools.partial(_pool_kernel, C=C),
        out_shape=jax.ShapeDtypeStruct((rows, Wo, C), BF16),
        grid_spec=pltpu.PrefetchScalarGridSpec(
            num_scalar_prefetch=0,
            grid=(rows // R,),
            in_specs=[pl.BlockSpec((R, 2, Wo, 2 * C),
                                   lambda i: (i, 0, 0, 0))],
            out_specs=pl.BlockSpec((R, Wo, C), lambda i: (i, 0, 0)),
        ),
        compiler_params=pltpu.CompilerParams(
            dimension_semantics=("parallel",),
            vmem_limit_bytes=48 * 1024 * 1024,
        ),
    )(x4)
    return out.reshape(B, Ho, Wo, C)


def _fc_kernel(w, xt, b, o, acc, *, kn, relu):
    k = pl.program_id(1)

    @pl.when(k == 0)
    def _():
        acc[...] = jnp.zeros_like(acc)

    acc[...] += jnp.dot(w[...], xt[...], preferred_element_type=F32)

    @pl.when(k == kn - 1)
    def _():
        y = acc[...] + b[...]
        if relu:
            y = jnp.maximum(y, 0.0)
        o[...] = y


def _fc(w, xt, b, *, relu, kblk):
    dout, din = w.shape
    nb = xt.shape[1]
    dp = _rup(dout, 16)
    wp = jnp.pad(w, ((0, dp - dout), (0, 0)))
    bp = jnp.pad(b, (0, dp - dout)).reshape(dp, 1)
    assert din % kblk == 0
    kn = din // kblk
    dh = dp // 2

    out = pl.pallas_call(
        functools.partial(_fc_kernel, kn=kn, relu=relu),
        out_shape=jax.ShapeDtypeStruct((dp, nb), F32),
        grid_spec=pltpu.PrefetchScalarGridSpec(
            num_scalar_prefetch=0,
            grid=(2, kn),
            in_specs=[
                pl.BlockSpec((dh, kblk), lambda j, k: (j, k)),
                pl.BlockSpec((kblk, nb), lambda j, k: (k, 0)),
                pl.BlockSpec((dh, 1), lambda j, k: (j, 0)),
            ],
            out_specs=pl.BlockSpec((dh, nb), lambda j, k: (j, 0)),
            scratch_shapes=[pltpu.VMEM((dh, nb), F32)],
        ),
        compiler_params=pltpu.CompilerParams(
            dimension_semantics=("parallel", "arbitrary"),
            vmem_limit_bytes=48 * 1024 * 1024,
        ),
    )(wp, xt, bp)
    return out[:dout]


def _fold_bn(conv_b, gamma, beta, mean, var, eps=1e-5):
    inv = lax.rsqrt(var + eps)
    scale = gamma * inv
    bias = beta + (conv_b - mean) * scale
    return scale, bias


def _pack3(y):
    B, H, W, C = y.shape
    yp = jnp.pad(y, ((0, 0), (0, 0), (1, 1), (0, 0)))
    xc = jnp.concatenate([yp[:, :, 0:W, :], yp[:, :, 1:W + 1, :],
                          yp[:, :, 2:W + 2, :]], axis=-1)
    xc = jnp.pad(xc, ((0, 0), (1, 1), (0, 0), (0, 0)))
    return xc.reshape(B * (H + 2) * W, 3 * C)


def _pad9(y):
    B, H, W, C = y.shape
    yp = jnp.pad(y, ((0, 0), (0, 2), (0, 2), (0, 0)))
    return yp.reshape(B * (H + 2) * (W + 2), C)


def _offs9(W):
    return [(dy - 1) * (W + 2) + (dx - 1) for dy in range(3)
            for dx in range(3)]


def _wt9(w):
    t = jnp.transpose(w, (2, 3, 1, 0))
    return t.reshape(9, w.shape[1], w.shape[0]).astype(BF16)


def _wquad(w):
    co, ci = w.shape[0], w.shape[1]
    wt = jnp.transpose(w, (2, 3, 1, 0))
    z = jnp.zeros((ci, co), wt.dtype)
    taps = []
    for dy in range(3):
        for q in (-1, 0, 1):
            rows = []
            for pin in range(4):
                cols = [wt[dy, 4 * q + pin - p + 1]
                        if -1 <= 4 * q + pin - p <= 1 else z
                        for p in range(4)]
                rows.append(jnp.concatenate(cols, axis=1))
            taps.append(jnp.concatenate(rows, axis=0))
    return jnp.stack(taps).astype(BF16)


def _unflat9(o, B, H, W):
    return o.reshape(B, H + 2, W + 2, o.shape[-1])[:, :H, :W]


def _wt3(w):
    t = jnp.transpose(w, (2, 3, 1, 0))
    return t.reshape(3, 3 * w.shape[1], w.shape[0]).astype(BF16)


def _unflat(o, B, H, W):
    return o.reshape(B, H + 2, W, o.shape[-1])[:, 1:H + 1]


_MBLK = {192: 4096, 384: 2048, 768: 2048, 1536: 1024}


def kernel(conv0_w, conv0_b, conv0_gamma, conv0_beta, conv0_mean, conv0_var, conv1_w, conv1_b, conv1_gamma, conv1_beta, conv1_mean, conv1_var, conv2_w, conv2_b, conv2_gamma, conv2_beta, conv2_mean, conv2_var, conv3_w, conv3_b, conv3_gamma, conv3_beta, conv3_mean, conv3_var, conv4_w, conv4_b, conv4_gamma, conv4_beta, conv4_mean, conv4_var, conv5_w, conv5_b, conv5_gamma, conv5_beta, conv5_mean, conv5_var, conv6_w, conv6_b, conv6_gamma, conv6_beta, conv6_mean, conv6_var, conv7_w, conv7_b, conv7_gamma, conv7_beta, conv7_mean, conv7_var, conv8_w, conv8_b, conv8_gamma, conv8_beta, conv8_mean, conv8_var, conv9_w, conv9_b, conv9_gamma, conv9_beta, conv9_mean, conv9_var, conv10_w, conv10_b, conv10_gamma, conv10_beta, conv10_mean, conv10_var, conv11_w, conv11_b, conv11_gamma, conv11_beta, conv11_mean, conv11_var, conv12_w, conv12_b, conv12_gamma, conv12_beta, conv12_mean, conv12_var, fc0_w, fc0_b, fc1_w, fc1_b, fc2_w, fc2_b, x):
    conv = [
        (conv0_w, conv0_b, conv0_gamma, conv0_beta, conv0_mean, conv0_var),
        (conv1_w, conv1_b, conv1_gamma, conv1_beta, conv1_mean, conv1_var),
        (conv2_w, conv2_b, conv2_gamma, conv2_beta, conv2_mean, conv2_var),
        (conv3_w, conv3_b, conv3_gamma, conv3_beta, conv3_mean, conv3_var),
        (conv4_w, conv4_b, conv4_gamma, conv4_beta, conv4_mean, conv4_var),
        (conv5_w, conv5_b, conv5_gamma, conv5_beta, conv5_mean, conv5_var),
        (conv6_w, conv6_b, conv6_gamma, conv6_beta, conv6_mean, conv6_var),
        (conv7_w, conv7_b, conv7_gamma, conv7_beta, conv7_mean, conv7_var),
        (conv8_w, conv8_b, conv8_gamma, conv8_beta, conv8_mean, conv8_var),
        (conv9_w, conv9_b, conv9_gamma, conv9_beta, conv9_mean, conv9_var),
        (conv10_w, conv10_b, conv10_gamma, conv10_beta, conv10_mean, conv10_var),
        (conv11_w, conv11_b, conv11_gamma, conv11_beta, conv11_mean, conv11_var),
        (conv12_w, conv12_b, conv12_gamma, conv12_beta, conv12_mean, conv12_var),
    ]
    pool_after = {1, 3, 6, 9, 12}

    B = x.shape[0]
    H = W = 224
    xp = jnp.pad(x.astype(BF16), ((0, 0), (0, 13), (0, 2), (0, 4)))
    xn = jnp.transpose(xp, (0, 2, 3, 1))
    xq0 = xn.reshape(B * 226 * 57, 64)
    offs_q = [(dy - 1) * 57 + q for dy in range(3) for q in (-1, 0, 1)]
    mask_q = (57, 56, 226, 224)
    s0, b0 = _fold_bn(*conv[0][1:])
    w0 = jnp.pad(conv0_w, ((0, 0), (0, 13), (0, 0), (0, 0)))
    o = _tapconv(xq0, _wquad(w0), jnp.tile(s0, 4), jnp.tile(b0, 4),
                 offsets=offs_q, mblk=4096, mask_hw=mask_q)

    s1, b1 = _fold_bn(*conv[1][1:])
    o = _tapconv(o, _wquad(conv[1][0]), jnp.tile(s1, 4), jnp.tile(b1, 4),
                 offsets=offs_q, mblk=4096, mask_hw=mask_q)
    y = _maxpool(o.reshape(B, 226, 228, 64))
    y = jnp.pad(y, ((0, 0), (0, 1), (0, 2), (0, 0)))

    s2, b2 = _fold_bn(*conv[2][1:])
    o = _tapconv(y.reshape(B * 114 * 29, 256), _wquad(conv[2][0]),
                 jnp.tile(s2, 4), jnp.tile(b2, 4),
                 offsets=[(dy - 1) * 29 + q for dy in range(3)
                          for q in (-1, 0, 1)], mblk=2048)
    y = o.reshape(B, 114, 116, 128)[:, :112, :112]

    for i in range(3, 13):
        w = conv[i][0]
        cin = w.shape[1]
        Bc, Hc, Wc, _ = y.shape
        xflat = _pack3(y)
        wt = _wt3(w)
        si, bi = _fold_bn(*conv[i][1:])
        o = _tapconv(xflat, wt, si, bi, offsets=[-Wc, 0, Wc],
                     mblk=_MBLK[3 * cin])
        y = _unflat(o, Bc, Hc, Wc)
        if i in pool_after:
            y = _maxpool(y)

    xt = jnp.transpose(y, (3, 1, 2, 0)).reshape(512 * 7 * 7, B).astype(F32)
    h = _fc(fc0_w, xt, fc0_b, relu=True, kblk=512)
    h = _fc(fc1_w, h, fc1_b, relu=True, kblk=1024)
    h = _fc(fc2_w, h, fc2_b, relu=False, kblk=512)
    return jnp.transpose(h)


# speedup vs baseline: 1.1044x; 1.1044x over previous
import functools

import jax
import jax.numpy as jnp
from jax import lax
from jax.experimental import pallas as pl
from jax.experimental.pallas import tpu as pltpu

BF16 = jnp.bfloat16
F32 = jnp.float32


def _rup(x, m):
    return (x + m - 1) // m * m


def _tapconv_kernel(xf, wt, sc, bi, o, xbuf, obuf, isem, osem,
                    *, mblk, offs, steps, mask_hw, lpack):
    j = pl.program_id(0)
    span = max(offs)
    rows = _rup(mblk + span, 8)
    base = j * steps

    def start_in(slot, gi):
        pltpu.make_async_copy(xf.at[pl.ds(gi * mblk, rows)],
                              xbuf.at[slot], isem.at[slot]).start()

    def wait_in(slot):
        pltpu.make_async_copy(xbuf.at[slot], xbuf.at[slot],
                              isem.at[slot]).wait()

    def start_out(slot, gi):
        pltpu.make_async_copy(obuf.at[slot],
                              o.at[pl.ds(gi * mblk, mblk)],
                              osem.at[slot]).start()

    def wait_out(slot):
        pltpu.make_async_copy(obuf.at[slot], obuf.at[slot],
                              osem.at[slot]).wait()

    start_in(0, base)

    def body(s, carry):
        cur = lax.rem(s, 2)
        nxt = lax.rem(s + 1, 2)

        @pl.when(s + 1 < steps)
        def _():
            start_in(nxt, base + s + 1)

        wait_in(cur)

        @pl.when(s >= 2)
        def _():
            wait_out(cur)

        xr = xbuf.at[cur]
        acc = None
        for t in range(len(offs) // lpack):
            if lpack == 1:
                xs = xr[pl.ds(offs[t], mblk), :]
            else:
                xs = jnp.concatenate(
                    [xr[pl.ds(offs[t * lpack + i], mblk), :]
                     for i in range(lpack)], axis=1)
            d = jnp.dot(xs, wt[t], preferred_element_type=F32)
            acc = d if acc is None else acc + d
        y = jnp.maximum(acc * sc[...] + bi[...], 0.0)
        if mask_hw is not None:
            colper, colvalid, rowper, rowvalid = mask_hw
            g = jax.lax.broadcasted_iota(jnp.int32, (mblk, 1), 0) \
                + (base + s) * mblk
            col = g % colper
            row = (g // colper) % rowper
            y = jnp.where((col < colvalid) & (row < rowvalid), y, 0.0)
        obuf.at[cur][...] = y.astype(obuf.dtype)
        start_out(cur, base + s)
        return carry

    lax.fori_loop(0, steps, body, 0)
    wait_out(lax.rem(steps - 2, 2))
    wait_out(lax.rem(steps - 1, 2))


def _tapconv(xflat, wt, scale, bias, *, offsets, mblk, mask_hw=None,
             lpack=1):
    N, K = xflat.shape
    ndots, _, cout = wt.shape
    assert ndots * lpack == len(offsets)
    mblk = min(mblk, max(N // 4 // 8 * 8, 8))
    lo = -min(offsets)
    offs = [o + lo for o in offsets]
    span = max(offs)
    ntot = _rup(N, 2 * mblk)
    steps = ntot // (2 * mblk)
    rows = _rup(mblk + span, 8)
    xp = jnp.pad(xflat, ((lo, (ntot - N) + (rows - mblk - lo)), (0, 0)))
    sc = scale.reshape(1, cout).astype(F32)
    bi = bias.reshape(1, cout).astype(F32)

    out = pl.pallas_call(
        functools.partial(_tapconv_kernel, mblk=mblk, offs=offs,
                          steps=steps, mask_hw=mask_hw, lpack=lpack),
        out_shape=jax.ShapeDtypeStruct((ntot, cout), BF16),
        grid_spec=pltpu.PrefetchScalarGridSpec(
            num_scalar_prefetch=0,
            grid=(2,),
            in_specs=[
                pl.BlockSpec(memory_space=pl.ANY),
                pl.BlockSpec(wt.shape, lambda j: (0, 0, 0)),
                pl.BlockSpec((1, cout), lambda j: (0, 0)),
                pl.BlockSpec((1, cout), lambda j: (0, 0)),
            ],
            out_specs=pl.BlockSpec(memory_space=pl.ANY),
            scratch_shapes=[
                pltpu.VMEM((2, rows, K), BF16),
                pltpu.VMEM((2, mblk, cout), BF16),
                pltpu.SemaphoreType.DMA((2,)),
                pltpu.SemaphoreType.DMA((2,)),
            ],
        ),
        compiler_params=pltpu.CompilerParams(
            dimension_semantics=("parallel",),
            vmem_limit_bytes=48 * 1024 * 1024,
        ),
    )(xp, wt, sc, bi)
    return out[:N]


def _pool_kernel(xr, o, *, C):
    m = jnp.maximum(xr[:, 0], xr[:, 1])
    o[...] = jnp.maximum(m[:, :, :C], m[:, :, C:])


def _maxpool(x):
    B, H, W, C = x.shape
    Ho, Wo = H // 2, W // 2
    rows = B * Ho
    x4 = x.reshape(rows, 2, Wo, 2 * C)
    R = 8
    for cand in (64, 32, 16):
        if rows % cand == 0:
            R = cand
            break
    out = pl.pallas_call(
        functools.partial(_pool_kernel, C=C),
        out_shape=jax.ShapeDtypeStruct((rows, Wo, C), BF16),
        grid_spec=pltpu.PrefetchScalarGridSpec(
            num_scalar_prefetch=0,
            grid=(rows // R,),
            in_specs=[pl.BlockSpec((R, 2, Wo, 2 * C),
                                   lambda i: (i, 0, 0, 0))],
            out_specs=pl.BlockSpec((R, Wo, C), lambda i: (i, 0, 0)),
        ),
        compiler_params=pltpu.CompilerParams(
            dimension_semantics=("parallel",),
            vmem_limit_bytes=48 * 1024 * 1024,
        ),
    )(x4)
    return out.reshape(B, Ho, Wo, C)


def _fc_kernel(w, xt, b, o, acc, *, kn, relu):
    k = pl.program_id(1)

    @pl.when(k == 0)
    def _():
        acc[...] = jnp.zeros_like(acc)

    acc[...] += jnp.dot(w[...], xt[...], preferred_element_type=F32)

    @pl.when(k == kn - 1)
    def _():
        y = acc[...] + b[...]
        if relu:
            y = jnp.maximum(y, 0.0)
        o[...] = y


def _fc(w, xt, b, *, relu, kblk):
    dout, din = w.shape
    nb = xt.shape[1]
    dp = _rup(dout, 16)
    wp = jnp.pad(w, ((0, dp - dout), (0, 0)))
    bp = jnp.pad(b, (0, dp - dout)).reshape(dp, 1)
    assert din % kblk == 0
    kn = din // kblk
    dh = dp // 2

    out = pl.pallas_call(
        functools.partial(_fc_kernel, kn=kn, relu=relu),
        out_shape=jax.ShapeDtypeStruct((dp, nb), F32),
        grid_spec=pltpu.PrefetchScalarGridSpec(
            num_scalar_prefetch=0,
            grid=(2, kn),
            in_specs=[
                pl.BlockSpec((dh, kblk), lambda j, k: (j, k)),
                pl.BlockSpec((kblk, nb), lambda j, k: (k, 0)),
                pl.BlockSpec((dh, 1), lambda j, k: (j, 0)),
            ],
            out_specs=pl.BlockSpec((dh, nb), lambda j, k: (j, 0)),
            scratch_shapes=[pltpu.VMEM((dh, nb), F32)],
        ),
        compiler_params=pltpu.CompilerParams(
            dimension_semantics=("parallel", "arbitrary"),
            vmem_limit_bytes=48 * 1024 * 1024,
        ),
    )(wp, xt, bp)
    return out[:dout]


def _fold_bn(conv_b, gamma, beta, mean, var, eps=1e-5):
    inv = lax.rsqrt(var + eps)
    scale = gamma * inv
    bias = beta + (conv_b - mean) * scale
    return scale, bias


def _pack3(y):
    B, H, W, C = y.shape
    yp = jnp.pad(y, ((0, 0), (0, 0), (1, 1), (0, 0)))
    xc = jnp.concatenate([yp[:, :, 0:W, :], yp[:, :, 1:W + 1, :],
                          yp[:, :, 2:W + 2, :]], axis=-1)
    xc = jnp.pad(xc, ((0, 0), (1, 1), (0, 0), (0, 0)))
    return xc.reshape(B * (H + 2) * W, 3 * C)


def _pad9(y):
    B, H, W, C = y.shape
    yp = jnp.pad(y, ((0, 0), (0, 2), (0, 2), (0, 0)))
    return yp.reshape(B * (H + 2) * (W + 2), C)


def _offs9(W):
    return [(dy - 1) * (W + 2) + (dx - 1) for dy in range(3)
            for dx in range(3)]


def _wt9(w):
    t = jnp.transpose(w, (2, 3, 1, 0))
    return t.reshape(9, w.shape[1], w.shape[0]).astype(BF16)


def _wquad(w):
    return _wpack(w, 4)


def _wpack(w, P):
    co, ci = w.shape[0], w.shape[1]
    wt = jnp.transpose(w, (2, 3, 1, 0))
    z = jnp.zeros((ci, co), wt.dtype)
    taps = []
    for dy in range(3):
        for q in (-1, 0, 1):
            rows = []
            for pin in range(P):
                cols = [wt[dy, P * q + pin - p + 1]
                        if -1 <= P * q + pin - p <= 1 else z
                        for p in range(P)]
                rows.append(jnp.concatenate(cols, axis=1))
            taps.append(jnp.concatenate(rows, axis=0))
    return jnp.stack(taps).astype(BF16)


def _unflat9(o, B, H, W):
    return o.reshape(B, H + 2, W + 2, o.shape[-1])[:, :H, :W]


def _wt3(w):
    t = jnp.transpose(w, (2, 3, 1, 0))
    return t.reshape(3, 3 * w.shape[1], w.shape[0]).astype(BF16)


def _unflat(o, B, H, W):
    return o.reshape(B, H + 2, W, o.shape[-1])[:, 1:H + 1]


_MBLK = {192: 4096, 384: 2048, 768: 2048, 1536: 1024}


def kernel(conv0_w, conv0_b, conv0_gamma, conv0_beta, conv0_mean, conv0_var, conv1_w, conv1_b, conv1_gamma, conv1_beta, conv1_mean, conv1_var, conv2_w, conv2_b, conv2_gamma, conv2_beta, conv2_mean, conv2_var, conv3_w, conv3_b, conv3_gamma, conv3_beta, conv3_mean, conv3_var, conv4_w, conv4_b, conv4_gamma, conv4_beta, conv4_mean, conv4_var, conv5_w, conv5_b, conv5_gamma, conv5_beta, conv5_mean, conv5_var, conv6_w, conv6_b, conv6_gamma, conv6_beta, conv6_mean, conv6_var, conv7_w, conv7_b, conv7_gamma, conv7_beta, conv7_mean, conv7_var, conv8_w, conv8_b, conv8_gamma, conv8_beta, conv8_mean, conv8_var, conv9_w, conv9_b, conv9_gamma, conv9_beta, conv9_mean, conv9_var, conv10_w, conv10_b, conv10_gamma, conv10_beta, conv10_mean, conv10_var, conv11_w, conv11_b, conv11_gamma, conv11_beta, conv11_mean, conv11_var, conv12_w, conv12_b, conv12_gamma, conv12_beta, conv12_mean, conv12_var, fc0_w, fc0_b, fc1_w, fc1_b, fc2_w, fc2_b, x):
    conv = [
        (conv0_w, conv0_b, conv0_gamma, conv0_beta, conv0_mean, conv0_var),
        (conv1_w, conv1_b, conv1_gamma, conv1_beta, conv1_mean, conv1_var),
        (conv2_w, conv2_b, conv2_gamma, conv2_beta, conv2_mean, conv2_var),
        (conv3_w, conv3_b, conv3_gamma, conv3_beta, conv3_mean, conv3_var),
        (conv4_w, conv4_b, conv4_gamma, conv4_beta, conv4_mean, conv4_var),
        (conv5_w, conv5_b, conv5_gamma, conv5_beta, conv5_mean, conv5_var),
        (conv6_w, conv6_b, conv6_gamma, conv6_beta, conv6_mean, conv6_var),
        (conv7_w, conv7_b, conv7_gamma, conv7_beta, conv7_mean, conv7_var),
        (conv8_w, conv8_b, conv8_gamma, conv8_beta, conv8_mean, conv8_var),
        (conv9_w, conv9_b, conv9_gamma, conv9_beta, conv9_mean, conv9_var),
        (conv10_w, conv10_b, conv10_gamma, conv10_beta, conv10_mean, conv10_var),
        (conv11_w, conv11_b, conv11_gamma, conv11_beta, conv11_mean, conv11_var),
        (conv12_w, conv12_b, conv12_gamma, conv12_beta, conv12_mean, conv12_var),
    ]
    pool_after = {1, 3, 6, 9, 12}

    B = x.shape[0]
    H = W = 224
    xn = jnp.transpose(x, (0, 2, 3, 1)).astype(BF16)
    xn = jnp.pad(xn, ((0, 0), (0, 2), (0, 4), (0, 13)))
    xq0 = xn.reshape(B * 226 * 57, 64)
    offs_q = [(dy - 1) * 57 + q for dy in range(3) for q in (-1, 0, 1)]
    mask_q = (57, 56, 226, 224)
    s0, b0 = _fold_bn(*conv[0][1:])
    w0 = jnp.pad(conv0_w, ((0, 0), (0, 13), (0, 0), (0, 0)))
    o = _tapconv(xq0, _wquad(w0), jnp.tile(s0, 4), jnp.tile(b0, 4),
                 offsets=offs_q, mblk=4096, mask_hw=mask_q)

    s1, b1 = _fold_bn(*conv[1][1:])
    o = _tapconv(o, _wquad(conv[1][0]), jnp.tile(s1, 4), jnp.tile(b1, 4),
                 offsets=offs_q, mblk=4096, mask_hw=mask_q)
    y = _maxpool(o.reshape(B, 226, 228, 64))
    y = jnp.pad(y, ((0, 0), (0, 1), (0, 2), (0, 0)))

    s2, b2 = _fold_bn(*conv[2][1:])
    o = _tapconv(y.reshape(B * 114 * 29, 256), _wquad(conv[2][0]),
                 jnp.tile(s2, 4), jnp.tile(b2, 4),
                 offsets=[(dy - 1) * 29 + q for dy in range(3)
                          for q in (-1, 0, 1)], mblk=2048,
                 mask_hw=(29, 28, 114, 112))

    s3, b3 = _fold_bn(*conv[3][1:])
    o = _tapconv(o.reshape(B * 114 * 58, 256), _wpack(conv[3][0], 2),
                 jnp.tile(s3, 2), jnp.tile(b3, 2),
                 offsets=[(dy - 1) * 58 + q for dy in range(3)
                          for q in (-1, 0, 1)], mblk=2048)
    y = _maxpool(o.reshape(B, 114, 116, 128))[:, :56, :56]

    for i in range(4, 13):
        w = conv[i][0]
        cin = w.shape[1]
        Bc, Hc, Wc, _ = y.shape
        xflat = _pack3(y)
        wt = _wt3(w)
        si, bi = _fold_bn(*conv[i][1:])
        o = _tapconv(xflat, wt, si, bi, offsets=[-Wc, 0, Wc],
                     mblk=_MBLK[3 * cin])
        y = _unflat(o, Bc, Hc, Wc)
        if i in pool_after:
            y = _maxpool(y)

    xt = jnp.transpose(y, (3, 1, 2, 0)).reshape(512 * 7 * 7, B).astype(F32)
    h = _fc(fc0_w, xt, fc0_b, relu=True, kblk=512)
    h = _fc(fc1_w, h, fc1_b, relu=True, kblk=1024)
    h = _fc(fc2_w, h, fc2_b, relu=False, kblk=512)
    return jnp.transpose(h)
